# initial kernel scaffold (unmeasured)
import jax
import jax.numpy as jnp
from jax import lax
from jax.experimental import pallas as pl
from jax.experimental.pallas import tpu as pltpu


def kernel(
    t,
):
    def body(*refs):
        pass

    out_shape = jax.ShapeDtypeStruct(..., jnp.float32)
    return pl.pallas_call(body, out_shape=out_shape)(...)


# baseline (device time: 57855 ns/iter reference)
import jax
import jax.numpy as jnp
from jax import lax
from jax.experimental import pallas as pl
from jax.experimental.pallas import tpu as pltpu

N_DEV = 8


def kernel(t):
    m, n = t.shape

    def body(t_ref, out_ref, comm_ref, send_sems, recv_sems):
        my = lax.axis_index("i")
        left = lax.rem(my + N_DEV - 1, N_DEV)
        right = lax.rem(my + 1, N_DEV)

        barrier_sem = pltpu.get_barrier_semaphore()
        pl.semaphore_signal(
            barrier_sem,
            inc=1,
            device_id=(left,),
            device_id_type=pl.DeviceIdType.MESH,
        )
        pl.semaphore_wait(barrier_sem, 1)

        comm_ref[0] = t_ref[...]
        out_ref[...] = t_ref[...]

        for h in range(N_DEV - 1):
            rdma = pltpu.make_async_remote_copy(
                src_ref=comm_ref.at[h],
                dst_ref=comm_ref.at[h + 1],
                send_sem=send_sems.at[h],
                recv_sem=recv_sems.at[h],
                device_id=(right,),
                device_id_type=pl.DeviceIdType.MESH,
            )
            rdma.start()
            rdma.wait()
            out_ref[...] += comm_ref[h + 1]

        s = out_ref[...]
        r = jnp.maximum(s, 0.0)
        out_ref[...] = jnp.tanh(s) * s * s + r * r * r

    return pl.pallas_call(
        body,
        out_shape=jax.ShapeDtypeStruct((m, n), jnp.float32),
        in_specs=[pl.BlockSpec(memory_space=pltpu.VMEM)],
        out_specs=pl.BlockSpec(memory_space=pltpu.VMEM),
        scratch_shapes=[
            pltpu.VMEM((N_DEV, m, n), t.dtype),
            pltpu.SemaphoreType.DMA((N_DEV - 1,)),
            pltpu.SemaphoreType.DMA((N_DEV - 1,)),
        ],
        compiler_params=pltpu.CompilerParams(collective_id=0),
    )(t)


# device time: 20128 ns/iter; 2.8744x vs baseline; 2.8744x over previous
import jax
import jax.numpy as jnp
from jax import lax
from jax.experimental import pallas as pl
from jax.experimental.pallas import tpu as pltpu

N_DEV = 8
N_STEPS = 3


def kernel(t):
    m, n = t.shape

    def body(t_ref, out_ref, send_bufs, recv_bufs, send_sems, recv_sems):
        my = lax.axis_index("i")

        barrier_sem = pltpu.get_barrier_semaphore()
        for k in range(N_STEPS):
            partner = my ^ (1 << k)
            pl.semaphore_signal(
                barrier_sem,
                inc=1,
                device_id=(partner,),
                device_id_type=pl.DeviceIdType.MESH,
            )
        pl.semaphore_wait(barrier_sem, N_STEPS)

        out_ref[...] = t_ref[...]

        for k in range(N_STEPS):
            partner = my ^ (1 << k)
            send_bufs[k] = out_ref[...].astype(jnp.bfloat16)
            rdma = pltpu.make_async_remote_copy(
                src_ref=send_bufs.at[k],
                dst_ref=recv_bufs.at[k],
                send_sem=send_sems.at[k],
                recv_sem=recv_sems.at[k],
                device_id=(partner,),
                device_id_type=pl.DeviceIdType.MESH,
            )
            rdma.start()
            rdma.wait()
            out_ref[...] += recv_bufs[k].astype(jnp.float32)

        s = out_ref[...]
        r = jnp.maximum(s, 0.0)
        out_ref[...] = jnp.tanh(s) * s * s + r * r * r

    return pl.pallas_call(
        body,
        out_shape=jax.ShapeDtypeStruct((m, n), jnp.float32),
        in_specs=[pl.BlockSpec(memory_space=pltpu.VMEM)],
        out_specs=pl.BlockSpec(memory_space=pltpu.VMEM),
        scratch_shapes=[
            pltpu.VMEM((N_STEPS, m, n), jnp.bfloat16),
            pltpu.VMEM((N_STEPS, m, n), jnp.bfloat16),
            pltpu.SemaphoreType.DMA((N_STEPS,)),
            pltpu.SemaphoreType.DMA((N_STEPS,)),
        ],
        compiler_params=pltpu.CompilerParams(collective_id=0),
    )(t)


# device time: 17204 ns/iter; 3.3629x vs baseline; 1.1700x over previous
import jax
import jax.numpy as jnp
from jax import lax
from jax.experimental import pallas as pl
from jax.experimental.pallas import tpu as pltpu

N_DEV = 8
N_STEPS = 3
N_HALF = 2


def kernel(t):
    m, n = t.shape
    hm = m // N_HALF

    def body(t_ref, out_ref, send_bufs, recv_bufs, send_sems, recv_sems):
        my = lax.axis_index("i")

        barrier_sem = pltpu.get_barrier_semaphore()
        for k in range(N_STEPS):
            pl.semaphore_signal(
                barrier_sem,
                inc=1,
                device_id=(my ^ (1 << k),),
                device_id_type=pl.DeviceIdType.MESH,
            )
        pl.semaphore_wait(barrier_sem, N_STEPS)

        rdmas = {}

        def start(k, h):
            r = pltpu.make_async_remote_copy(
                src_ref=send_bufs.at[k, h],
                dst_ref=recv_bufs.at[k, h],
                send_sem=send_sems.at[k, h],
                recv_sem=recv_sems.at[k, h],
                device_id=(my ^ (1 << k),),
                device_id_type=pl.DeviceIdType.MESH,
            )
            rdmas[(k, h)] = r
            r.start()

        for h in range(N_HALF):
            sl = pl.ds(h * hm, hm)
            out_ref[sl, :] = t_ref[sl, :]
            send_bufs[0, h] = t_ref[sl, :].astype(jnp.bfloat16)
            start(0, h)

        for k in range(N_STEPS):
            for h in range(N_HALF):
                sl = pl.ds(h * hm, hm)
                rdmas[(k, h)].wait()
                out_ref[sl, :] += recv_bufs[k, h].astype(jnp.float32)
                if k + 1 < N_STEPS:
                    send_bufs[k + 1, h] = out_ref[sl, :].astype(jnp.bfloat16)
                    start(k + 1, h)
                else:
                    s = out_ref[sl, :]
                    r = jnp.maximum(s, 0.0)
                    out_ref[sl, :] = jnp.tanh(s) * s * s + r * r * r

    return pl.pallas_call(
        body,
        out_shape=jax.ShapeDtypeStruct((m, n), jnp.float32),
        in_specs=[pl.BlockSpec(memory_space=pltpu.VMEM)],
        out_specs=pl.BlockSpec(memory_space=pltpu.VMEM),
        scratch_shapes=[
            pltpu.VMEM((N_STEPS, N_HALF, hm, n), jnp.bfloat16),
            pltpu.VMEM((N_STEPS, N_HALF, hm, n), jnp.bfloat16),
            pltpu.SemaphoreType.DMA((N_STEPS, N_HALF)),
            pltpu.SemaphoreType.DMA((N_STEPS, N_HALF)),
        ],
        compiler_params=pltpu.CompilerParams(collective_id=0),
    )(t)


# device time: 16236 ns/iter; 3.5634x vs baseline; 1.0596x over previous
import jax
import jax.numpy as jnp
from jax import lax
from jax.experimental import pallas as pl
from jax.experimental.pallas import tpu as pltpu

N_DEV = 8
MASKS = (1, 3, 4)
N_STEPS = len(MASKS)
N_HALF = 2


def kernel(t):
    m, n = t.shape
    hm = m // N_HALF

    def body(t_ref, out_ref, send_bufs, recv_bufs, send_sems, recv_sems):
        my = lax.axis_index("i")

        barrier_sem = pltpu.get_barrier_semaphore()
        for mask in MASKS:
            pl.semaphore_signal(
                barrier_sem,
                inc=1,
                device_id=(my ^ mask,),
                device_id_type=pl.DeviceIdType.MESH,
            )
        pl.semaphore_wait(barrier_sem, N_STEPS)

        rdmas = {}

        def start(k, h):
            r = pltpu.make_async_remote_copy(
                src_ref=send_bufs.at[k, h],
                dst_ref=recv_bufs.at[k, h],
                send_sem=send_sems.at[k, h],
                recv_sem=recv_sems.at[k, h],
                device_id=(my ^ MASKS[k],),
                device_id_type=pl.DeviceIdType.MESH,
            )
            rdmas[(k, h)] = r
            r.start()

        for h in range(N_HALF):
            send_bufs[0, h] = t_ref[pl.ds(h * hm, hm), :].astype(jnp.bfloat16)
            start(0, h)

        for k in range(N_STEPS):
            for h in range(N_HALF):
                rdmas[(k, h)].wait()
                if k + 1 < N_STEPS:
                    send_bufs[k + 1, h] = send_bufs[k, h] + recv_bufs[k, h]
                    start(k + 1, h)
                else:
                    s = (send_bufs[k, h] + recv_bufs[k, h]).astype(jnp.float32)
                    r = jnp.maximum(s, 0.0)
                    out_ref[pl.ds(h * hm, hm), :] = (
                        jnp.tanh(s) * s * s + r * r * r
                    )

    return pl.pallas_call(
        body,
        out_shape=jax.ShapeDtypeStruct((m, n), jnp.float32),
        in_specs=[pl.BlockSpec(memory_space=pltpu.VMEM)],
        out_specs=pl.BlockSpec(memory_space=pltpu.VMEM),
        scratch_shapes=[
            pltpu.VMEM((N_STEPS, N_HALF, hm, n), jnp.bfloat16),
            pltpu.VMEM((N_STEPS, N_HALF, hm, n), jnp.bfloat16),
            pltpu.SemaphoreType.DMA((N_STEPS, N_HALF)),
            pltpu.SemaphoreType.DMA((N_STEPS, N_HALF)),
        ],
        compiler_params=pltpu.CompilerParams(collective_id=0),
    )(t)


# device time: 15050 ns/iter; 3.8442x vs baseline; 1.0788x over previous
import jax
import jax.numpy as jnp
from jax import lax
from jax.experimental import pallas as pl
from jax.experimental.pallas import tpu as pltpu

N_DEV = 8
MASKS = (1, 3, 4)
N_STEPS = len(MASKS)
N_HALF = 4


def kernel(t):
    m, n = t.shape
    hm = m // N_HALF

    def body(t_ref, out_ref, send_bufs, recv_bufs, send_sems, recv_sems):
        my = lax.axis_index("i")

        barrier_sem = pltpu.get_barrier_semaphore()
        for mask in MASKS:
            pl.semaphore_signal(
                barrier_sem,
                inc=1,
                device_id=(my ^ mask,),
                device_id_type=pl.DeviceIdType.MESH,
            )
        pl.semaphore_wait(barrier_sem, N_STEPS)

        rdmas = {}

        def start(k, h):
            r = pltpu.make_async_remote_copy(
                src_ref=send_bufs.at[k, h],
                dst_ref=recv_bufs.at[k, h],
                send_sem=send_sems.at[k, h],
                recv_sem=recv_sems.at[k, h],
                device_id=(my ^ MASKS[k],),
                device_id_type=pl.DeviceIdType.MESH,
            )
            rdmas[(k, h)] = r
            r.start()

        for h in range(N_HALF):
            send_bufs[0, h] = t_ref[pl.ds(h * hm, hm), :].astype(jnp.bfloat16)
            start(0, h)

        for k in range(N_STEPS):
            for h in range(N_HALF):
                rdmas[(k, h)].wait()
                if k + 1 < N_STEPS:
                    send_bufs[k + 1, h] = send_bufs[k, h] + recv_bufs[k, h]
                    start(k + 1, h)
                else:
                    s = (send_bufs[k, h] + recv_bufs[k, h]).astype(jnp.float32)
                    r = jnp.maximum(s, 0.0)
                    out_ref[pl.ds(h * hm, hm), :] = (
                        jnp.tanh(s) * s * s + r * r * r
                    )

    return pl.pallas_call(
        body,
        out_shape=jax.ShapeDtypeStruct((m, n), jnp.float32),
        in_specs=[pl.BlockSpec(memory_space=pltpu.VMEM)],
        out_specs=pl.BlockSpec(memory_space=pltpu.VMEM),
        scratch_shapes=[
            pltpu.VMEM((N_STEPS, N_HALF, hm, n), jnp.bfloat16),
            pltpu.VMEM((N_STEPS, N_HALF, hm, n), jnp.bfloat16),
            pltpu.SemaphoreType.DMA((N_STEPS, N_HALF)),
            pltpu.SemaphoreType.DMA((N_STEPS, N_HALF)),
        ],
        compiler_params=pltpu.CompilerParams(collective_id=0),
    )(t)
